# baseline (speedup 1.0000x reference)
import jax
import jax.numpy as jnp
from jax.experimental import pallas as pl

_N = 10000
_D_IN = 128
_D_H = 32
_D_OUT = 7
_D_CAT = _D_IN + _D_H
_BLOCK = 2000


def _dot(a, b):
    return jax.lax.dot_general(
        a, b, (((1,), (0,)), ((), ())), preferred_element_type=jnp.float32
    )


def _cell_body(x_ref, h_ref, wz_ref, bz_ref, wr_ref, br_ref, wh_ref, bh_ref,
               wl_ref, bl_ref, out_ref, hnew_ref):
    x = x_ref[...]
    h = h_ref[...]
    wz = wz_ref[0] + wz_ref[1]
    wr = wr_ref[0] + wr_ref[1]
    wh = wh_ref[0] + wh_ref[1]
    z = jax.nn.sigmoid(_dot(x, wz[:_D_IN]) + _dot(h, wz[_D_IN:]) + bz_ref[...])
    r = jax.nn.sigmoid(_dot(x, wr[:_D_IN]) + _dot(h, wr[_D_IN:]) + br_ref[...])
    ht = jnp.tanh(_dot(x, wh[:_D_IN]) + _dot(h * r, wh[_D_IN:]) + bh_ref[...])
    hn = z * h + (1.0 - z) * ht
    hnew_ref[...] = hn
    out_ref[...] = _dot(jnp.maximum(hn, 0.0), wl_ref[...]) + bl_ref[...]


def kernel(x, edge_index, edge_weight, h, W_z, b_z, W_r, b_r, W_h, b_h,
           W_lin, b_lin):
    del edge_index, edge_weight

    wz = W_z.reshape(2, _D_CAT, _D_H)
    wr = W_r.reshape(2, _D_CAT, _D_H)
    wh = W_h.reshape(2, _D_CAT, _D_H)
    bz = b_z.reshape(1, _D_H)
    br = b_r.reshape(1, _D_H)
    bh = b_h.reshape(1, _D_H)
    bl = b_lin.reshape(1, _D_OUT)

    grid = (_N // _BLOCK,)
    row_spec = lambda d: pl.BlockSpec((_BLOCK, d), lambda i: (i, 0))
    full2 = lambda s: pl.BlockSpec(s, lambda i: (0, 0))
    full3 = lambda s: pl.BlockSpec(s, lambda i: (0, 0, 0))

    out, hnew = pl.pallas_call(
        _cell_body,
        grid=grid,
        in_specs=[
            row_spec(_D_IN),
            row_spec(_D_H),
            full3((2, _D_CAT, _D_H)),
            full2((1, _D_H)),
            full3((2, _D_CAT, _D_H)),
            full2((1, _D_H)),
            full3((2, _D_CAT, _D_H)),
            full2((1, _D_H)),
            full2((_D_H, _D_OUT)),
            full2((1, _D_OUT)),
        ],
        out_specs=[
            row_spec(_D_OUT),
            row_spec(_D_H),
        ],
        out_shape=[
            jax.ShapeDtypeStruct((_N, _D_OUT), jnp.float32),
            jax.ShapeDtypeStruct((_N, _D_H), jnp.float32),
        ],
    )(x, h, wz, bz, wr, br, wh, bh, W_lin, bl)
    return out, hnew


# speedup vs baseline: 1.0025x; 1.0025x over previous
import jax
import jax.numpy as jnp
from jax.experimental import pallas as pl
from jax.experimental.pallas import tpu as pltpu

_N = 10000
_D_IN = 128
_D_H = 32
_D_OUT = 7
_D_CAT = _D_IN + _D_H
_BLOCK = 2000


def _dot(a, b):
    return jax.lax.dot_general(
        a, b, (((1,), (0,)), ((), ())), preferred_element_type=jnp.float32
    )


def _cell_body(x_ref, h_ref, wz_ref, bz_ref, wr_ref, br_ref, wh_ref, bh_ref,
               wl_ref, bl_ref, out_ref, hnew_ref):
    x = x_ref[...]
    h = h_ref[...]
    wz = wz_ref[0] + wz_ref[1]
    wr = wr_ref[0] + wr_ref[1]
    wh = wh_ref[0] + wh_ref[1]
    z = jax.nn.sigmoid(_dot(x, wz[:_D_IN]) + _dot(h, wz[_D_IN:]) + bz_ref[...])
    r = jax.nn.sigmoid(_dot(x, wr[:_D_IN]) + _dot(h, wr[_D_IN:]) + br_ref[...])
    ht = jnp.tanh(_dot(x, wh[:_D_IN]) + _dot(h * r, wh[_D_IN:]) + bh_ref[...])
    hn = z * h + (1.0 - z) * ht
    hnew_ref[...] = hn
    out_ref[...] = _dot(jnp.maximum(hn, 0.0), wl_ref[...]) + bl_ref[...]


def kernel(x, edge_index, edge_weight, h, W_z, b_z, W_r, b_r, W_h, b_h,
           W_lin, b_lin):
    del edge_index, edge_weight

    wz = W_z.reshape(2, _D_CAT, _D_H)
    wr = W_r.reshape(2, _D_CAT, _D_H)
    wh = W_h.reshape(2, _D_CAT, _D_H)
    bz = b_z.reshape(1, _D_H)
    br = b_r.reshape(1, _D_H)
    bh = b_h.reshape(1, _D_H)
    bl = b_lin.reshape(1, _D_OUT)

    grid = (_N // _BLOCK,)
    row_spec = lambda d: pl.BlockSpec((_BLOCK, d), lambda i: (i, 0))
    full2 = lambda s: pl.BlockSpec(s, lambda i: (0, 0))
    full3 = lambda s: pl.BlockSpec(s, lambda i: (0, 0, 0))

    out, hnew = pl.pallas_call(
        _cell_body,
        grid=grid,
        in_specs=[
            row_spec(_D_IN),
            row_spec(_D_H),
            full3((2, _D_CAT, _D_H)),
            full2((1, _D_H)),
            full3((2, _D_CAT, _D_H)),
            full2((1, _D_H)),
            full3((2, _D_CAT, _D_H)),
            full2((1, _D_H)),
            full2((_D_H, _D_OUT)),
            full2((1, _D_OUT)),
        ],
        out_specs=[
            row_spec(_D_OUT),
            row_spec(_D_H),
        ],
        out_shape=[
            jax.ShapeDtypeStruct((_N, _D_OUT), jnp.float32),
            jax.ShapeDtypeStruct((_N, _D_H), jnp.float32),
        ],
        compiler_params=pltpu.CompilerParams(
            dimension_semantics=("parallel",),
        ),
    )(x, h, wz, bz, wr, br, wh, bh, W_lin, bl)
    return out, hnew
